# baseline (speedup 1.0000x reference)
import functools

import numpy as np
import jax
import jax.numpy as jnp
from jax import lax
from jax.experimental import pallas as pl
from jax.experimental.pallas import tpu as pltpu
from jax.experimental.pallas import tpu_sc as plsc

_NC = 2
_NS = 16
_NW = _NC * _NS
_IDX_CHUNK = 100


def _pe_table(seq_len, dmodel):
    position = np.arange(seq_len, dtype=np.float32)[:, None]
    div_term = np.exp(
        np.arange(0, dmodel, 2, dtype=np.float32)
        * (-np.log(np.float32(10000.0)) / np.float32(dmodel))
    )
    pe = np.zeros((seq_len, dmodel), dtype=np.float32)
    pe[:, 0::2] = np.sin(position * div_term)
    pe[:, 1::2] = np.cos(position * div_term)
    return jnp.asarray(pe)


@functools.lru_cache(maxsize=None)
def _make_sc_kernel(n_batch, seq_len, dmodel):
    assert n_batch % _NW == 0
    assert seq_len % _IDX_CHUNK == 0
    n_chunks = seq_len // _IDX_CHUNK
    seq_per_w = n_batch // _NW
    mesh = plsc.VectorSubcoreMesh(core_axis_name="c", subcore_axis_name="s")

    @functools.partial(
        pl.kernel,
        out_type=jax.ShapeDtypeStruct((n_batch * seq_len, dmodel), jnp.float32),
        mesh=mesh,
        compiler_params=pltpu.CompilerParams(use_tc_tiling_on_sc=False),
        scratch_types=[
            pltpu.VMEM((n_chunks, _IDX_CHUNK), jnp.int32),
            pltpu.VMEM((seq_len, dmodel), jnp.float32),
            pltpu.SemaphoreType.DMA,
        ],
    )
    def run(x_hbm, pe_hbm, table_hbm, out_hbm, idx_v, rows_v, sem):
        wid = lax.axis_index("s") * _NC + lax.axis_index("c")

        def one_seq(b, carry):
            seq = wid * seq_per_w + b
            pltpu.sync_copy(x_hbm.at[pl.ds(seq * n_chunks, n_chunks)], idx_v)
            pltpu.sync_copy(pe_hbm, rows_v)
            copies = [
                pltpu.async_copy(
                    table_hbm.at[idx_v.at[j]],
                    rows_v.at[pl.ds(j * _IDX_CHUNK, _IDX_CHUNK)],
                    sem,
                    add=True,
                )
                for j in range(n_chunks)
            ]
            for c in copies:
                c.wait()
            pltpu.sync_copy(rows_v, out_hbm.at[pl.ds(seq * seq_len, seq_len)])
            return carry

        lax.fori_loop(0, seq_per_w, one_seq, 0)

    return run


def kernel(x, emb_table):
    n_batch, seq_len = x.shape
    _, dmodel = emb_table.shape
    pe = _pe_table(seq_len, dmodel)
    x_flat = x.astype(jnp.int32).reshape(
        n_batch * seq_len // _IDX_CHUNK, _IDX_CHUNK
    )
    out = _make_sc_kernel(n_batch, seq_len, dmodel)(x_flat, pe, emb_table)
    return out.reshape(n_batch, seq_len, dmodel)


# speedup vs baseline: 1.2986x; 1.2986x over previous
import functools

import numpy as np
import jax
import jax.numpy as jnp
from jax import lax
from jax.experimental import pallas as pl
from jax.experimental.pallas import tpu as pltpu
from jax.experimental.pallas import tpu_sc as plsc

_NC = 2
_NS = 16
_NW = _NC * _NS
_IDX_CHUNK = 100
_NB = 4


def _pe_table(seq_len, dmodel):
    position = np.arange(seq_len, dtype=np.float32)[:, None]
    div_term = np.exp(
        np.arange(0, dmodel, 2, dtype=np.float32)
        * (-np.log(np.float32(10000.0)) / np.float32(dmodel))
    )
    pe = np.zeros((seq_len, dmodel), dtype=np.float32)
    pe[:, 0::2] = np.sin(position * div_term)
    pe[:, 1::2] = np.cos(position * div_term)
    return jnp.asarray(pe)


@functools.lru_cache(maxsize=None)
def _make_sc_kernel(n_batch, seq_len, dmodel):
    assert n_batch % _NW == 0
    assert seq_len % _IDX_CHUNK == 0
    n_chunks = seq_len // _IDX_CHUNK
    seq_per_w = n_batch // _NW
    n_outer = seq_per_w // _NB
    assert seq_per_w % _NB == 0 and n_outer >= 2
    mesh = plsc.VectorSubcoreMesh(core_axis_name="c", subcore_axis_name="s")

    @functools.partial(
        pl.kernel,
        out_type=jax.ShapeDtypeStruct((n_batch * seq_len, dmodel), jnp.float32),
        mesh=mesh,
        compiler_params=pltpu.CompilerParams(use_tc_tiling_on_sc=False),
        scratch_types=[
            pltpu.VMEM_SHARED((seq_len, dmodel), jnp.float32),
            pltpu.VMEM((seq_per_w * n_chunks, _IDX_CHUNK), jnp.int32),
            pltpu.VMEM((_NB, seq_len, dmodel), jnp.float32),
            pltpu.SemaphoreType.DMA((_NB,)),
            pltpu.SemaphoreType.DMA((_NB,)),
            pltpu.SemaphoreType.DMA((_NB,)),
        ],
    )
    def run(x_hbm, pe_hbm, table_hbm, out_hbm, pe_v, idx_v, rows_v, psem,
            gsem, osem):
        sid = lax.axis_index("s")
        wid = sid * _NC + lax.axis_index("c")
        seq0 = wid * seq_per_w

        @pl.when(sid == 0)
        def _():
            pltpu.sync_copy(pe_hbm, pe_v)

        pltpu.sync_copy(
            x_hbm.at[pl.ds(seq0 * n_chunks, seq_per_w * n_chunks)], idx_v
        )
        plsc.subcore_barrier()

        def fire_pe(k):
            pltpu.async_copy(pe_v, rows_v.at[k], psem.at[k])

        def wait_pe(k):
            pltpu.make_async_copy(pe_v, rows_v.at[k], psem.at[k]).wait()

        def fire_gather(s, k):
            for j in range(n_chunks):
                pltpu.async_copy(
                    table_hbm.at[idx_v.at[s * n_chunks + j]],
                    rows_v.at[k].at[pl.ds(j * _IDX_CHUNK, _IDX_CHUNK)],
                    gsem.at[k], add=True,
                )

        def wait_gather(k):
            pltpu.make_async_copy(
                table_hbm.at[pl.ds(0, seq_len)], rows_v.at[k], gsem.at[k]
            ).wait()

        def fire_out(s, k):
            pltpu.async_copy(
                rows_v.at[k],
                out_hbm.at[pl.ds((seq0 + s) * seq_len, seq_len)],
                osem.at[k],
            )

        def wait_out(k):
            pltpu.make_async_copy(
                rows_v.at[k], out_hbm.at[pl.ds(0, seq_len)], osem.at[k]
            ).wait()

        for kk in range(_NB):
            fire_pe(kk)
            if kk >= 1:
                wait_gather(kk - 1)
                fire_out(kk - 1, kk - 1)
            wait_pe(kk)
            fire_gather(kk, kk)

        def outer(g, carry):
            s_base = g * _NB
            for kk in range(_NB):
                s = s_base + kk
                wait_out(kk)
                fire_pe(kk)
                kp = (kk - 1) % _NB
                wait_gather(kp)
                fire_out(s - 1, kp)
                wait_pe(kk)
                fire_gather(s, kk)
            return carry

        lax.fori_loop(1, n_outer, outer, 0)

        wait_gather(_NB - 1)
        fire_out(seq_per_w - 1, _NB - 1)
        for kk in range(_NB):
            wait_out(kk)

    return run


def kernel(x, emb_table):
    n_batch, seq_len = x.shape
    _, dmodel = emb_table.shape
    pe = _pe_table(seq_len, dmodel)
    x_flat = x.astype(jnp.int32).reshape(
        n_batch * seq_len // _IDX_CHUNK, _IDX_CHUNK
    )
    out = _make_sc_kernel(n_batch, seq_len, dmodel)(x_flat, pe, emb_table)
    return out.reshape(n_batch, seq_len, dmodel)


# speedup vs baseline: 1.3002x; 1.0012x over previous
import functools

import numpy as np
import jax
import jax.numpy as jnp
from jax import lax
from jax.experimental import pallas as pl
from jax.experimental.pallas import tpu as pltpu
from jax.experimental.pallas import tpu_sc as plsc

_NC = 2
_NS = 16
_NW = _NC * _NS
_IDX_CHUNK = 100
_NB = 4


def _pe_table(seq_len, dmodel):
    position = np.arange(seq_len, dtype=np.float32)[:, None]
    div_term = np.exp(
        np.arange(0, dmodel, 2, dtype=np.float32)
        * (-np.log(np.float32(10000.0)) / np.float32(dmodel))
    )
    pe = np.zeros((seq_len, dmodel), dtype=np.float32)
    pe[:, 0::2] = np.sin(position * div_term)
    pe[:, 1::2] = np.cos(position * div_term)
    return jnp.asarray(pe)


@functools.lru_cache(maxsize=None)
def _make_sc_kernel(n_batch, seq_len, dmodel):
    assert n_batch % _NW == 0
    assert seq_len % _IDX_CHUNK == 0
    n_chunks = seq_len // _IDX_CHUNK
    seq_per_w = n_batch // _NW
    n_outer = seq_per_w // _NB
    assert seq_per_w % _NB == 0 and n_outer >= 2
    mesh = plsc.VectorSubcoreMesh(core_axis_name="c", subcore_axis_name="s")

    @functools.partial(
        pl.kernel,
        out_type=jax.ShapeDtypeStruct((n_batch, seq_len, dmodel), jnp.float32),
        mesh=mesh,
        compiler_params=pltpu.CompilerParams(use_tc_tiling_on_sc=False),
        scratch_types=[
            pltpu.VMEM_SHARED((seq_len, dmodel), jnp.float32),
            pltpu.VMEM((seq_per_w * n_chunks, _IDX_CHUNK), jnp.int32),
            pltpu.VMEM((_NB, seq_len, dmodel), jnp.float32),
            pltpu.SemaphoreType.DMA((_NB,)),
            pltpu.SemaphoreType.DMA((_NB,)),
            pltpu.SemaphoreType.DMA((_NB,)),
        ],
    )
    def run(x_hbm, pe_hbm, table_hbm, out_hbm, pe_v, idx_v, rows_v, psem,
            gsem, osem):
        sid = lax.axis_index("s")
        wid = sid * _NC + lax.axis_index("c")
        seq0 = wid * seq_per_w

        @pl.when(sid == 0)
        def _():
            pltpu.sync_copy(pe_hbm, pe_v)

        pltpu.sync_copy(
            x_hbm.at[pl.ds(seq0 * n_chunks, seq_per_w * n_chunks)], idx_v
        )
        plsc.subcore_barrier()

        def fire_pe(k):
            pltpu.async_copy(pe_v, rows_v.at[k], psem.at[k])

        def wait_pe(k):
            pltpu.make_async_copy(pe_v, rows_v.at[k], psem.at[k]).wait()

        def fire_gather(s, k):
            for j in range(n_chunks):
                pltpu.async_copy(
                    table_hbm.at[idx_v.at[s * n_chunks + j]],
                    rows_v.at[k].at[pl.ds(j * _IDX_CHUNK, _IDX_CHUNK)],
                    gsem.at[k], add=True,
                )

        def wait_gather(k):
            pltpu.make_async_copy(
                table_hbm.at[pl.ds(0, seq_len)], rows_v.at[k], gsem.at[k]
            ).wait()

        def fire_out(s, k):
            pltpu.async_copy(rows_v.at[k], out_hbm.at[seq0 + s], osem.at[k])

        def wait_out(k):
            pltpu.make_async_copy(
                rows_v.at[k], out_hbm.at[0], osem.at[k]
            ).wait()

        for kk in range(_NB):
            fire_pe(kk)
            if kk >= 1:
                wait_gather(kk - 1)
                fire_out(kk - 1, kk - 1)
            wait_pe(kk)
            fire_gather(kk, kk)

        def outer(g, carry):
            s_base = g * _NB
            for kk in range(_NB):
                s = s_base + kk
                wait_out(kk)
                fire_pe(kk)
                kp = (kk - 1) % _NB
                wait_gather(kp)
                fire_out(s - 1, kp)
                wait_pe(kk)
                fire_gather(s, kk)
            return carry

        lax.fori_loop(1, n_outer, outer, 0)

        wait_gather(_NB - 1)
        fire_out(seq_per_w - 1, _NB - 1)
        for kk in range(_NB):
            wait_out(kk)

    return run


def kernel(x, emb_table):
    n_batch, seq_len = x.shape
    _, dmodel = emb_table.shape
    pe = _pe_table(seq_len, dmodel)
    x_flat = x.astype(jnp.int32).reshape(
        n_batch * seq_len // _IDX_CHUNK, _IDX_CHUNK
    )
    return _make_sc_kernel(n_batch, seq_len, dmodel)(x_flat, pe, emb_table)


# speedup vs baseline: 1.6014x; 1.2317x over previous
import functools

import numpy as np
import jax
import jax.numpy as jnp
from jax import lax
from jax.experimental import pallas as pl
from jax.experimental.pallas import tpu as pltpu
from jax.experimental.pallas import tpu_sc as plsc
from jax.experimental import layout as jex_layout

_NC = 2
_NS = 16
_NW = _NC * _NS
_IDX_CHUNK = 100
_NB = 4


def _pe_table(seq_len, dmodel):
    position = np.arange(seq_len, dtype=np.float32)[:, None]
    div_term = np.exp(
        np.arange(0, dmodel, 2, dtype=np.float32)
        * (-np.log(np.float32(10000.0)) / np.float32(dmodel))
    )
    pe = np.zeros((seq_len, dmodel), dtype=np.float32)
    pe[:, 0::2] = np.sin(position * div_term)
    pe[:, 1::2] = np.cos(position * div_term)
    return jnp.asarray(pe)


@functools.lru_cache(maxsize=None)
def _make_sc_kernel(n_batch, seq_len, dmodel):
    assert n_batch % _NW == 0
    assert seq_len % _IDX_CHUNK == 0
    n_chunks = seq_len // _IDX_CHUNK
    seq_per_w = n_batch // _NW
    n_outer = seq_per_w // _NB
    assert seq_per_w % _NB == 0 and n_outer >= 2
    mesh = plsc.VectorSubcoreMesh(core_axis_name="c", subcore_axis_name="s")

    @functools.partial(
        pl.kernel,
        out_type=jax.ShapeDtypeStruct((n_batch, seq_len, dmodel), jnp.float32),
        mesh=mesh,
        compiler_params=pltpu.CompilerParams(use_tc_tiling_on_sc=False),
        scratch_types=[
            pltpu.VMEM_SHARED((seq_len, dmodel), jnp.float32),
            pltpu.VMEM((seq_per_w * n_chunks, _IDX_CHUNK), jnp.int32),
            pltpu.VMEM((_NB, seq_len, dmodel), jnp.float32),
            pltpu.SemaphoreType.DMA((_NB,)),
            pltpu.SemaphoreType.DMA((_NB,)),
            pltpu.SemaphoreType.DMA((_NB,)),
        ],
    )
    def run(x_hbm, pe_hbm, table_hbm, out_hbm, pe_v, idx_v, rows_v, psem,
            gsem, osem):
        sid = lax.axis_index("s")
        wid = sid * _NC + lax.axis_index("c")
        seq0 = wid * seq_per_w

        @pl.when(sid == 0)
        def _():
            pltpu.sync_copy(pe_hbm, pe_v)

        pltpu.sync_copy(
            x_hbm.at[pl.ds(seq0 * n_chunks, seq_per_w * n_chunks)], idx_v
        )
        plsc.subcore_barrier()

        def fire_pe(k):
            pltpu.async_copy(pe_v, rows_v.at[k], psem.at[k])

        def wait_pe(k):
            pltpu.make_async_copy(pe_v, rows_v.at[k], psem.at[k]).wait()

        def fire_gather(s, k):
            for j in range(n_chunks):
                pltpu.async_copy(
                    table_hbm.at[idx_v.at[s * n_chunks + j]],
                    rows_v.at[k].at[pl.ds(j * _IDX_CHUNK, _IDX_CHUNK)],
                    gsem.at[k], add=True,
                )

        def wait_gather(k):
            pltpu.make_async_copy(
                table_hbm.at[pl.ds(0, seq_len)], rows_v.at[k], gsem.at[k]
            ).wait()

        def fire_out(s, k):
            pltpu.async_copy(rows_v.at[k], out_hbm.at[seq0 + s], osem.at[k])

        def wait_out(k):
            pltpu.make_async_copy(
                rows_v.at[k], out_hbm.at[0], osem.at[k]
            ).wait()

        for kk in range(_NB):
            fire_pe(kk)
            if kk >= 1:
                wait_gather(kk - 1)
                fire_out(kk - 1, kk - 1)
            wait_pe(kk)
            fire_gather(kk, kk)

        def outer(g, carry):
            s_base = g * _NB
            for kk in range(_NB):
                s = s_base + kk
                wait_out(kk)
                fire_pe(kk)
                kp = (kk - 1) % _NB
                wait_gather(kp)
                fire_out(s - 1, kp)
                wait_pe(kk)
                fire_gather(s, kk)
            return carry

        lax.fori_loop(1, n_outer, outer, 0)

        wait_gather(_NB - 1)
        fire_out(seq_per_w - 1, _NB - 1)
        for kk in range(_NB):
            wait_out(kk)

    return run


def kernel(x, emb_table):
    n_batch, seq_len = x.shape
    _, dmodel = emb_table.shape
    pe = _pe_table(seq_len, dmodel)
    x_flat = x.astype(jnp.int32).reshape(
        n_batch * seq_len // _IDX_CHUNK, _IDX_CHUNK
    )
    emb_lin = jex_layout.with_layout_constraint(
        emb_table, jex_layout.Layout((0, 1), tiling=())
    )
    return _make_sc_kernel(n_batch, seq_len, dmodel)(x_flat, pe, emb_lin)
